# baseline (speedup 1.0000x reference)
import functools

import jax
import jax.numpy as jnp
from jax import lax
from jax.experimental import pallas as pl
from jax.experimental.pallas import tpu as pltpu
from jax.experimental.pallas import tpu_sc as plsc

NUM_CORES = 2
NUM_SUBCORES = 16
NUM_WORKERS = NUM_CORES * NUM_SUBCORES

NUM_ROWS = 100000
BATCH = 16384
EMBED_DIM = 64
ROW_PAD = 2 * EMBED_DIM
SPLIT = 1
B_SPLIT = BATCH // SPLIT
B_PER_W = B_SPLIT // NUM_WORKERS
CHUNK = 128
NCHUNK = B_PER_W // CHUNK

TR_BLK = 12800
MLP_BLK = 16384


def _transpose_body(ut_t, it_t, e1_ref, e2_ref, tp):
    dn = (((0,), (0,)), ((), ()))
    u = lax.dot_general(ut_t[...], e1_ref[...], dn, preferred_element_type=jnp.float32)
    i = lax.dot_general(it_t[...], e2_ref[...], dn, preferred_element_type=jnp.float32)
    tp[...] = u + i


def _tc_transpose(user_table_t, item_table_t, e1, e2):
    grid = (pl.cdiv(NUM_ROWS, TR_BLK),)
    return pl.pallas_call(
        _transpose_body,
        grid=grid,
        in_specs=[
            pl.BlockSpec((EMBED_DIM, TR_BLK), lambda i: (0, i)),
            pl.BlockSpec((EMBED_DIM, TR_BLK), lambda i: (0, i)),
            pl.BlockSpec((EMBED_DIM, ROW_PAD), lambda i: (0, 0)),
            pl.BlockSpec((EMBED_DIM, ROW_PAD), lambda i: (0, 0)),
        ],
        out_specs=pl.BlockSpec((TR_BLK, ROW_PAD), lambda i: (i, 0)),
        out_shape=jax.ShapeDtypeStruct((NUM_ROWS, ROW_PAD), jnp.float32),
    )(user_table_t, item_table_t, e1, e2)


def _gather_body(tp, uid_r, iid_r, x, uidx, iidx, urows, irows, sem):
    wid = lax.axis_index("s") * NUM_CORES + lax.axis_index("c")
    base = wid * B_PER_W

    pltpu.sync_copy(uid_r.at[wid], uidx)
    pltpu.sync_copy(iid_r.at[wid], iidx)

    def start(j):
        return (
            pltpu.async_copy(tp.at[uidx.at[j]], urows.at[j % 2], sem),
            pltpu.async_copy(tp.at[iidx.at[j]], irows.at[j % 2], sem),
        )

    pending = start(0)
    for j in range(NCHUNK):
        nxt = start(j + 1) if j + 1 < NCHUNK else None
        for c in pending:
            c.wait()
        row0 = base + j * CHUNK
        pltpu.sync_copy(
            urows.at[j % 2, :, pl.ds(0, EMBED_DIM)],
            x.at[pl.ds(row0, CHUNK), pl.ds(0, EMBED_DIM)],
        )
        pltpu.sync_copy(
            irows.at[j % 2, :, pl.ds(EMBED_DIM, EMBED_DIM)],
            x.at[pl.ds(row0, CHUNK), pl.ds(EMBED_DIM, EMBED_DIM)],
        )
        pending = nxt


def _sc_gather(tp, uid_r, iid_r):
    mesh = plsc.VectorSubcoreMesh(
        core_axis_name="c",
        subcore_axis_name="s",
        num_cores=NUM_CORES,
        num_subcores=NUM_SUBCORES,
    )
    f = pl.kernel(
        _gather_body,
        out_type=jax.ShapeDtypeStruct((B_SPLIT, ROW_PAD), jnp.float32),
        mesh=mesh,
        scratch_types=[
            pltpu.VMEM((NCHUNK, CHUNK), jnp.int32),
            pltpu.VMEM((NCHUNK, CHUNK), jnp.int32),
            pltpu.VMEM((2, CHUNK, ROW_PAD), jnp.float32),
            pltpu.VMEM((2, CHUNK, ROW_PAD), jnp.float32),
            pltpu.SemaphoreType.DMA,
        ],
        compiler_params=pltpu.CompilerParams(use_tc_tiling_on_sc=False),
    )
    return f(tp, uid_r, iid_r)


def _mlp_body(x_ref, w1_ref, b1_ref, w2_ref, b2_ref, w3_ref, b3_ref, o_ref):
    dn = (((1,), (1,)), ((), ()))
    h = lax.dot_general(x_ref[...], w1_ref[...], dn, preferred_element_type=jnp.float32)
    h = jnp.maximum(h + b1_ref[...], 0.0)
    h = lax.dot_general(h, w2_ref[...], dn, preferred_element_type=jnp.float32)
    h = jnp.maximum(h + b2_ref[...], 0.0)
    o = jnp.sum(h * w3_ref[...], axis=1, keepdims=True)
    o_ref[...] = o + b3_ref[0]


def _tc_mlp(x, W1, b1, W2, b2, W3, b3):
    grid = (B_SPLIT // MLP_BLK,)
    full = lambda shape: pl.BlockSpec(shape, lambda i: (0, 0))
    return pl.pallas_call(
        _mlp_body,
        grid=grid,
        in_specs=[
            pl.BlockSpec((MLP_BLK, ROW_PAD), lambda i: (i, 0)),
            full(W1.shape),
            full(b1.shape),
            full(W2.shape),
            full(b2.shape),
            full(W3.shape),
            pl.BlockSpec(memory_space=pltpu.SMEM),
        ],
        out_specs=pl.BlockSpec((MLP_BLK, 1), lambda i: (i, 0)),
        out_shape=jax.ShapeDtypeStruct((B_SPLIT, 1), jnp.float32),
    )(x, W1, b1, W2, b2, W3, b3)


@jax.jit
def kernel(user_id, item_id, user_table, item_table, W1, b1, W2, b2, W3, b3):
    uid_r = user_id.astype(jnp.int32).reshape(SPLIT, NUM_WORKERS, NCHUNK, CHUNK)
    iid_r = item_id.astype(jnp.int32).reshape(SPLIT, NUM_WORKERS, NCHUNK, CHUNK)
    e1 = jnp.eye(EMBED_DIM, ROW_PAD, dtype=jnp.float32)
    e2 = jnp.eye(EMBED_DIM, ROW_PAD, k=EMBED_DIM, dtype=jnp.float32)
    tp = _tc_transpose(user_table.T, item_table.T, e1, e2)
    outs = []
    for h in range(SPLIT):
        x = _sc_gather(tp, uid_r[h], iid_r[h])
        outs.append(
            _tc_mlp(x, W1, b1.reshape(1, -1), W2, b2.reshape(1, -1), W3, b3)
        )
    return jnp.concatenate(outs, axis=0)


# speedup vs baseline: 1.0121x; 1.0121x over previous
import functools

import jax
import jax.numpy as jnp
from jax import lax
from jax.experimental import pallas as pl
from jax.experimental.pallas import tpu as pltpu
from jax.experimental.pallas import tpu_sc as plsc

NUM_CORES = 2
NUM_SUBCORES = 16
NUM_WORKERS = NUM_CORES * NUM_SUBCORES

NUM_ROWS = 100000
BATCH = 16384
EMBED_DIM = 64
ROW_PAD = 2 * EMBED_DIM
SPLIT = 1
B_SPLIT = BATCH // SPLIT
B_PER_W = B_SPLIT // NUM_WORKERS
CHUNK = 128
NCHUNK = B_PER_W // CHUNK

TR_BLK = 25600
MLP_BLK = 8192


def _transpose_body(ut_t, it_t, e1_ref, e2_ref, tp):
    dn = (((0,), (0,)), ((), ()))
    u = lax.dot_general(ut_t[...], e1_ref[...], dn, preferred_element_type=jnp.float32)
    i = lax.dot_general(it_t[...], e2_ref[...], dn, preferred_element_type=jnp.float32)
    tp[...] = u + i


def _tc_transpose(user_table_t, item_table_t, e1, e2):
    grid = (pl.cdiv(NUM_ROWS, TR_BLK),)
    return pl.pallas_call(
        _transpose_body,
        grid=grid,
        in_specs=[
            pl.BlockSpec((EMBED_DIM, TR_BLK), lambda i: (0, i)),
            pl.BlockSpec((EMBED_DIM, TR_BLK), lambda i: (0, i)),
            pl.BlockSpec((EMBED_DIM, ROW_PAD), lambda i: (0, 0)),
            pl.BlockSpec((EMBED_DIM, ROW_PAD), lambda i: (0, 0)),
        ],
        out_specs=pl.BlockSpec((TR_BLK, ROW_PAD), lambda i: (i, 0)),
        out_shape=jax.ShapeDtypeStruct((NUM_ROWS, ROW_PAD), jnp.float32),
        compiler_params=pltpu.CompilerParams(
            vmem_limit_bytes=100 * 1024 * 1024,
        ),
    )(user_table_t, item_table_t, e1, e2)


def _gather_body(tp, uid_r, iid_r, x, uidx, iidx, urows, irows, sem):
    wid = lax.axis_index("s") * NUM_CORES + lax.axis_index("c")
    base = wid * B_PER_W

    pltpu.sync_copy(uid_r.at[wid], uidx)
    pltpu.sync_copy(iid_r.at[wid], iidx)

    def start(j):
        return (
            pltpu.async_copy(tp.at[uidx.at[j]], urows.at[j % 2], sem),
            pltpu.async_copy(tp.at[iidx.at[j]], irows.at[j % 2], sem),
        )

    pending = start(0)
    for j in range(NCHUNK):
        nxt = start(j + 1) if j + 1 < NCHUNK else None
        for c in pending:
            c.wait()
        row0 = base + j * CHUNK
        pltpu.sync_copy(
            urows.at[j % 2, :, pl.ds(0, EMBED_DIM)],
            x.at[pl.ds(row0, CHUNK), pl.ds(0, EMBED_DIM)],
        )
        pltpu.sync_copy(
            irows.at[j % 2, :, pl.ds(EMBED_DIM, EMBED_DIM)],
            x.at[pl.ds(row0, CHUNK), pl.ds(EMBED_DIM, EMBED_DIM)],
        )
        pending = nxt


def _sc_gather(tp, uid_r, iid_r):
    mesh = plsc.VectorSubcoreMesh(
        core_axis_name="c",
        subcore_axis_name="s",
        num_cores=NUM_CORES,
        num_subcores=NUM_SUBCORES,
    )
    f = pl.kernel(
        _gather_body,
        out_type=jax.ShapeDtypeStruct((B_SPLIT, ROW_PAD), jnp.float32),
        mesh=mesh,
        scratch_types=[
            pltpu.VMEM((NCHUNK, CHUNK), jnp.int32),
            pltpu.VMEM((NCHUNK, CHUNK), jnp.int32),
            pltpu.VMEM((2, CHUNK, ROW_PAD), jnp.float32),
            pltpu.VMEM((2, CHUNK, ROW_PAD), jnp.float32),
            pltpu.SemaphoreType.DMA,
        ],
        compiler_params=pltpu.CompilerParams(use_tc_tiling_on_sc=False),
    )
    return f(tp, uid_r, iid_r)


def _mlp_body(x_ref, w1_ref, b1_ref, w2_ref, b2_ref, w3_ref, b3_ref, o_ref):
    dn = (((1,), (1,)), ((), ()))
    h = lax.dot_general(x_ref[...], w1_ref[...], dn, preferred_element_type=jnp.float32)
    h = jnp.maximum(h + b1_ref[...], 0.0)
    h = lax.dot_general(h, w2_ref[...], dn, preferred_element_type=jnp.float32)
    h = jnp.maximum(h + b2_ref[...], 0.0)
    o = jnp.sum(h * w3_ref[...], axis=1, keepdims=True)
    o_ref[...] = o + b3_ref[0]


def _tc_mlp(x, W1, b1, W2, b2, W3, b3):
    grid = (B_SPLIT // MLP_BLK,)
    full = lambda shape: pl.BlockSpec(shape, lambda i: (0, 0))
    return pl.pallas_call(
        _mlp_body,
        grid=grid,
        in_specs=[
            pl.BlockSpec((MLP_BLK, ROW_PAD), lambda i: (i, 0)),
            full(W1.shape),
            full(b1.shape),
            full(W2.shape),
            full(b2.shape),
            full(W3.shape),
            pl.BlockSpec(memory_space=pltpu.SMEM),
        ],
        out_specs=pl.BlockSpec((MLP_BLK, 1), lambda i: (i, 0)),
        out_shape=jax.ShapeDtypeStruct((B_SPLIT, 1), jnp.float32),
    )(x, W1, b1, W2, b2, W3, b3)


@jax.jit
def kernel(user_id, item_id, user_table, item_table, W1, b1, W2, b2, W3, b3):
    uid_r = user_id.astype(jnp.int32).reshape(SPLIT, NUM_WORKERS, NCHUNK, CHUNK)
    iid_r = item_id.astype(jnp.int32).reshape(SPLIT, NUM_WORKERS, NCHUNK, CHUNK)
    e1 = jnp.eye(EMBED_DIM, ROW_PAD, dtype=jnp.float32)
    e2 = jnp.eye(EMBED_DIM, ROW_PAD, k=EMBED_DIM, dtype=jnp.float32)
    tp = _tc_transpose(user_table.T, item_table.T, e1, e2)
    outs = []
    for h in range(SPLIT):
        x = _sc_gather(tp, uid_r[h], iid_r[h])
        outs.append(
            _tc_mlp(x, W1, b1.reshape(1, -1), W2, b2.reshape(1, -1), W3, b3)
        )
    return jnp.concatenate(outs, axis=0)


# speedup vs baseline: 1.0215x; 1.0093x over previous
import functools

import jax
import jax.numpy as jnp
from jax import lax
from jax.experimental import pallas as pl
from jax.experimental.pallas import tpu as pltpu
from jax.experimental.pallas import tpu_sc as plsc

NUM_CORES = 2
NUM_SUBCORES = 16
NUM_WORKERS = NUM_CORES * NUM_SUBCORES

NUM_ROWS = 100000
BATCH = 16384
EMBED_DIM = 64
ROW_PAD = 2 * EMBED_DIM
SPLIT = 1
B_SPLIT = BATCH // SPLIT
B_PER_W = B_SPLIT // NUM_WORKERS
CHUNK = 128
NCHUNK = B_PER_W // CHUNK

TR_BLK = 12800
MLP_BLK = 8192


def _transpose_body(ut_t, it_t, e1_ref, e2_ref, tp):
    dn = (((0,), (0,)), ((), ()))
    u = lax.dot_general(ut_t[...], e1_ref[...], dn, preferred_element_type=jnp.float32)
    i = lax.dot_general(it_t[...], e2_ref[...], dn, preferred_element_type=jnp.float32)
    tp[...] = u + i


def _tc_transpose(user_table_t, item_table_t, e1, e2):
    grid = (pl.cdiv(NUM_ROWS, TR_BLK),)
    return pl.pallas_call(
        _transpose_body,
        grid=grid,
        in_specs=[
            pl.BlockSpec((EMBED_DIM, TR_BLK), lambda i: (0, i)),
            pl.BlockSpec((EMBED_DIM, TR_BLK), lambda i: (0, i)),
            pl.BlockSpec((EMBED_DIM, ROW_PAD), lambda i: (0, 0)),
            pl.BlockSpec((EMBED_DIM, ROW_PAD), lambda i: (0, 0)),
        ],
        out_specs=pl.BlockSpec((TR_BLK, ROW_PAD), lambda i: (i, 0)),
        out_shape=jax.ShapeDtypeStruct((NUM_ROWS, ROW_PAD), jnp.float32),
        compiler_params=pltpu.CompilerParams(
            vmem_limit_bytes=100 * 1024 * 1024,
        ),
    )(user_table_t, item_table_t, e1, e2)


def _gather_body(tp, uid_r, iid_r, x, uidx, iidx, urows, irows, sem):
    wid = lax.axis_index("s") * NUM_CORES + lax.axis_index("c")
    base = wid * B_PER_W

    pltpu.sync_copy(uid_r.at[wid], uidx)
    pltpu.sync_copy(iid_r.at[wid], iidx)

    def start(j):
        return (
            pltpu.async_copy(tp.at[uidx.at[j]], urows.at[j % 2], sem),
            pltpu.async_copy(tp.at[iidx.at[j]], irows.at[j % 2], sem),
        )

    pending = start(0)
    for j in range(NCHUNK):
        nxt = start(j + 1) if j + 1 < NCHUNK else None
        for c in pending:
            c.wait()
        row0 = base + j * CHUNK
        pltpu.sync_copy(
            urows.at[j % 2, :, pl.ds(0, EMBED_DIM)],
            x.at[pl.ds(row0, CHUNK), pl.ds(0, EMBED_DIM)],
        )
        pltpu.sync_copy(
            irows.at[j % 2, :, pl.ds(EMBED_DIM, EMBED_DIM)],
            x.at[pl.ds(row0, CHUNK), pl.ds(EMBED_DIM, EMBED_DIM)],
        )
        pending = nxt


def _sc_gather(tp, uid_r, iid_r):
    mesh = plsc.VectorSubcoreMesh(
        core_axis_name="c",
        subcore_axis_name="s",
        num_cores=NUM_CORES,
        num_subcores=NUM_SUBCORES,
    )
    f = pl.kernel(
        _gather_body,
        out_type=jax.ShapeDtypeStruct((B_SPLIT, ROW_PAD), jnp.float32),
        mesh=mesh,
        scratch_types=[
            pltpu.VMEM((NCHUNK, CHUNK), jnp.int32),
            pltpu.VMEM((NCHUNK, CHUNK), jnp.int32),
            pltpu.VMEM((2, CHUNK, ROW_PAD), jnp.float32),
            pltpu.VMEM((2, CHUNK, ROW_PAD), jnp.float32),
            pltpu.SemaphoreType.DMA,
        ],
        compiler_params=pltpu.CompilerParams(use_tc_tiling_on_sc=False),
    )
    return f(tp, uid_r, iid_r)


def _mlp_body(x_ref, w1_ref, b1_ref, w2_ref, b2_ref, w3_ref, b3_ref, o_ref):
    dn = (((1,), (1,)), ((), ()))
    h = lax.dot_general(x_ref[...], w1_ref[...], dn, preferred_element_type=jnp.float32)
    h = jnp.maximum(h + b1_ref[...], 0.0)
    h = lax.dot_general(h, w2_ref[...], dn, preferred_element_type=jnp.float32)
    h = jnp.maximum(h + b2_ref[...], 0.0)
    o = jnp.sum(h * w3_ref[...], axis=1, keepdims=True)
    o_ref[...] = o + b3_ref[0]


def _tc_mlp(x, W1, b1, W2, b2, W3, b3):
    grid = (B_SPLIT // MLP_BLK,)
    full = lambda shape: pl.BlockSpec(shape, lambda i: (0, 0))
    return pl.pallas_call(
        _mlp_body,
        grid=grid,
        in_specs=[
            pl.BlockSpec((MLP_BLK, ROW_PAD), lambda i: (i, 0)),
            full(W1.shape),
            full(b1.shape),
            full(W2.shape),
            full(b2.shape),
            full(W3.shape),
            pl.BlockSpec(memory_space=pltpu.SMEM),
        ],
        out_specs=pl.BlockSpec((MLP_BLK, 1), lambda i: (i, 0)),
        out_shape=jax.ShapeDtypeStruct((B_SPLIT, 1), jnp.float32),
    )(x, W1, b1, W2, b2, W3, b3)


@jax.jit
def kernel(user_id, item_id, user_table, item_table, W1, b1, W2, b2, W3, b3):
    uid_r = user_id.astype(jnp.int32).reshape(SPLIT, NUM_WORKERS, NCHUNK, CHUNK)
    iid_r = item_id.astype(jnp.int32).reshape(SPLIT, NUM_WORKERS, NCHUNK, CHUNK)
    e1 = jnp.eye(EMBED_DIM, ROW_PAD, dtype=jnp.float32)
    e2 = jnp.eye(EMBED_DIM, ROW_PAD, k=EMBED_DIM, dtype=jnp.float32)
    tp = _tc_transpose(user_table.T, item_table.T, e1, e2)
    outs = []
    for h in range(SPLIT):
        x = _sc_gather(tp, uid_r[h], iid_r[h])
        outs.append(
            _tc_mlp(x, W1, b1.reshape(1, -1), W2, b2.reshape(1, -1), W3, b3)
        )
    return jnp.concatenate(outs, axis=0)
